# baseline (speedup 1.0000x reference)
import jax
import jax.numpy as jnp
from jax.experimental import pallas as pl
from jax.experimental.pallas import tpu as pltpu

_GLOBAL_WEIGHT = 1.0
_LOCAL_WEIGHT = 2.0

_ROWS = 4096
_COLS = 1024
_BLOCK_ROWS = 512


def _body(p_ref, o_ref, gm_ref, lm_ref, out_ref):
    i = pl.program_id(0)

    d = p_ref[...] - o_ref[...]
    d2 = d * d
    lm = lm_ref[...]
    gm_only = jnp.logical_and(gm_ref[...], jnp.logical_not(lm))

    gs = jnp.sum(jnp.where(gm_only, d2, 0.0))
    gc = jnp.sum(gm_only.astype(jnp.float32))
    ls = jnp.sum(jnp.where(lm, d2, 0.0))
    lc = jnp.sum(lm.astype(jnp.float32))

    @pl.when(i == 0)
    def _init():
        out_ref[0] = 0.0
        out_ref[1] = 0.0
        out_ref[2] = 0.0
        out_ref[3] = 0.0

    out_ref[0] += gs
    out_ref[1] += gc
    out_ref[2] += ls
    out_ref[3] += lc


def kernel(predicted_image, original_image, global_mask, local_mask):
    p = predicted_image.reshape(_ROWS, _COLS)
    o = original_image.reshape(_ROWS, _COLS)
    gm = global_mask.reshape(_ROWS, _COLS)
    lm = local_mask.reshape(_ROWS, _COLS)

    grid = (_ROWS // _BLOCK_ROWS,)
    in_spec = pl.BlockSpec((_BLOCK_ROWS, _COLS), lambda i: (i, 0))

    sums = pl.pallas_call(
        _body,
        grid=grid,
        in_specs=[in_spec, in_spec, in_spec, in_spec],
        out_specs=pl.BlockSpec(memory_space=pltpu.SMEM),
        out_shape=jax.ShapeDtypeStruct((4,), jnp.float32),
    )(p, o, gm, lm)

    global_loss = sums[0] / (sums[1] + 1e-08)
    local_loss = sums[2] / (sums[3] + 1e-08)
    total_loss = _GLOBAL_WEIGHT * global_loss + _LOCAL_WEIGHT * local_loss
    return (total_loss, global_loss, local_loss)


# speedup vs baseline: 1.1847x; 1.1847x over previous
import jax
import jax.numpy as jnp
from jax.experimental import pallas as pl
from jax.experimental.pallas import tpu as pltpu

_GLOBAL_WEIGHT = 1.0
_LOCAL_WEIGHT = 2.0

_ROWS = 4096
_COLS = 1024
_BLOCK_ROWS = 256


def _body(p_ref, o_ref, gm_ref, lm_ref, out_ref):
    i = pl.program_id(0)

    d = p_ref[...] - o_ref[...]
    d2 = d * d
    lm = lm_ref[...] != 0
    gm_only = jnp.logical_and(gm_ref[...] != 0, jnp.logical_not(lm))

    gs = jnp.sum(jnp.where(gm_only, d2, 0.0))
    gc = jnp.sum(gm_only.astype(jnp.float32))
    ls = jnp.sum(jnp.where(lm, d2, 0.0))
    lc = jnp.sum(lm.astype(jnp.float32))

    @pl.when(i == 0)
    def _init():
        out_ref[0] = 0.0
        out_ref[1] = 0.0
        out_ref[2] = 0.0
        out_ref[3] = 0.0

    out_ref[0] += gs
    out_ref[1] += gc
    out_ref[2] += ls
    out_ref[3] += lc


def kernel(predicted_image, original_image, global_mask, local_mask):
    p = predicted_image.reshape(_ROWS, _COLS)
    o = original_image.reshape(_ROWS, _COLS)
    gm = global_mask.view(jnp.int8).reshape(_ROWS, _COLS)
    lm = local_mask.view(jnp.int8).reshape(_ROWS, _COLS)

    grid = (_ROWS // _BLOCK_ROWS,)
    in_spec = pl.BlockSpec((_BLOCK_ROWS, _COLS), lambda i: (i, 0))

    sums = pl.pallas_call(
        _body,
        grid=grid,
        in_specs=[in_spec, in_spec, in_spec, in_spec],
        out_specs=pl.BlockSpec(memory_space=pltpu.SMEM),
        out_shape=jax.ShapeDtypeStruct((4,), jnp.float32),
    )(p, o, gm, lm)

    global_loss = sums[0] / (sums[1] + 1e-08)
    local_loss = sums[2] / (sums[3] + 1e-08)
    total_loss = _GLOBAL_WEIGHT * global_loss + _LOCAL_WEIGHT * local_loss
    return (total_loss, global_loss, local_loss)
